# baseline (speedup 1.0000x reference)
import functools

import jax
import jax.numpy as jnp
from jax.experimental import pallas as pl

K_TOT = 32
DIL = 2
B = 4
N = 4096
D = 64
QB = 512


def _dist_body(xq_ref, xk_ref, out_ref):
    q = xq_ref[0]
    k = xk_ref[0]
    inner = jax.lax.dot_general(
        q, k, (((1,), (1,)), ((), ())),
        preferred_element_type=jnp.float32,
    )
    x_inner = -2.0 * inner
    qsq = jnp.sum(q * q, axis=-1, keepdims=True)
    ksq = jnp.sum(k * k, axis=-1, keepdims=True)
    adj = qsq + x_inner + ksq.T
    out_ref[0] = -adj


def _neg_adj(xb):
    return pl.pallas_call(
        _dist_body,
        grid=(B, N // QB),
        in_specs=[
            pl.BlockSpec((1, QB, D), lambda b, i: (b, i, 0)),
            pl.BlockSpec((1, N, D), lambda b, i: (b, 0, 0)),
        ],
        out_specs=pl.BlockSpec((1, QB, N), lambda b, i: (b, i, 0)),
        out_shape=jax.ShapeDtypeStruct((B, N, N), jnp.float32),
    )(xb, xb)


def kernel(x, batch):
    del batch
    xb = x.reshape(B, N, D)
    neg_adj = _neg_adj(xb).reshape(B, N, N)
    val, nn_idx = jax.lax.top_k(neg_adj, K_TOT)
    start = (jnp.arange(B, dtype=nn_idx.dtype) * N).reshape(B, 1, 1)
    nn_idx = (nn_idx + start).reshape(1, -1)
    val = val.reshape(1, -1)
    center = jnp.repeat(jnp.arange(B * N, dtype=nn_idx.dtype), K_TOT).reshape(1, -1)
    edge_index = jnp.concatenate([nn_idx, center], axis=0)[:, ::DIL]
    return edge_index, val


# speedup vs baseline: 6.5940x; 6.5940x over previous
import functools

import jax
import jax.numpy as jnp
from jax import lax
from jax.experimental import pallas as pl
from jax.experimental.pallas import tpu as pltpu
from jax.experimental.pallas import tpu_sc as plsc

K_TOT = 32
DIL = 2
B = 4
N = 4096
D = 64
QB = 512

NWORK = 32
RPW = (B * N) // NWORK
G = 8
NEG_INF = float("-inf")


def _dist_body(xq_ref, xk_ref, out_ref):
    q = xq_ref[0]
    k = xk_ref[0]
    inner = jax.lax.dot_general(
        q, k, (((1,), (1,)), ((), ())),
        preferred_element_type=jnp.float32,
    )
    x_inner = -2.0 * inner
    qsq = jnp.sum(q * q, axis=-1, keepdims=True)
    ksq = jnp.sum(k * k, axis=-1, keepdims=True)
    adj = qsq + x_inner + ksq.T
    out_ref[0] = -adj


def _neg_adj(xb):
    return pl.pallas_call(
        _dist_body,
        grid=(B, N // QB),
        in_specs=[
            pl.BlockSpec((1, QB, D), lambda b, i: (b, i, 0)),
            pl.BlockSpec((1, N, D), lambda b, i: (b, 0, 0)),
        ],
        out_specs=pl.BlockSpec((1, QB, N), lambda b, i: (b, i, 0)),
        out_shape=jax.ShapeDtypeStruct((B, N, N), jnp.float32),
    )(xb, xb)


_MESH = plsc.VectorSubcoreMesh(core_axis_name="c", subcore_axis_name="s")
_VPR = N // 16


@functools.partial(
    pl.kernel,
    out_type=(
        jax.ShapeDtypeStruct((B * N * K_TOT,), jnp.float32),
        jax.ShapeDtypeStruct((B * N * K_TOT,), jnp.int32),
    ),
    mesh=_MESH,
    compiler_params=pltpu.CompilerParams(needs_layout_passes=False),
    scratch_types=[
        pltpu.VMEM((G * N,), jnp.float32),
        pltpu.VMEM((N + 16,), jnp.float32),
        pltpu.VMEM((N + 16,), jnp.int32),
        pltpu.VMEM((RPW * K_TOT,), jnp.float32),
        pltpu.VMEM((RPW * K_TOT,), jnp.int32),
    ],
)
def _topk_sc(neg_hbm, val_out, idx_out, inbuf, cand_v, cand_i, outv, outi):
    cid = lax.axis_index("c")
    sid = lax.axis_index("s")
    wid = sid * 2 + cid
    row0 = wid * RPW
    lane = lax.iota(jnp.int32, 16)
    ninf16 = jnp.full((16,), NEG_INF, jnp.float32)
    zero16 = jnp.zeros((16,), jnp.int32)

    def do_row(rr, slot):
        rbase = rr * N

        def scan_a(i, carry):
            m1, m2 = carry
            v = inbuf[pl.ds(pl.multiple_of(rbase + i * 16, 16), 16)]
            m2 = jnp.maximum(m2, jnp.minimum(m1, v))
            m1 = jnp.maximum(m1, v)
            return m1, m2

        m1, m2 = lax.fori_loop(0, _VPR, scan_a, (ninf16, ninf16))
        t = jnp.min(m2)

        def scan_b(i, off):
            v = inbuf[pl.ds(pl.multiple_of(rbase + i * 16, 16), 16)]
            mask = v >= t
            idxv = lane + i * 16
            plsc.store_compressed(cand_v.at[pl.ds(off, 16)], v, mask=mask)
            plsc.store_compressed(cand_i.at[pl.ds(off, 16)], idxv, mask=mask)
            return off + jnp.sum(mask.astype(jnp.int32))

        cnt = lax.fori_loop(0, _VPR, scan_b, jnp.int32(0))
        cand_v[pl.ds(cnt, 16)] = ninf16

        def merge(j, carry):
            b0v, b0i, b1v, b1i = carry
            cv = cand_v[pl.ds(pl.multiple_of(j * 16, 16), 16)]
            ci = cand_i[pl.ds(pl.multiple_of(j * 16, 16), 16)]
            cv, ci = plsc.sort_key_val(cv, ci, descending=True)
            rcv = lax.rev(cv, (0,))
            rci = lax.rev(ci, (0,))
            keep = b1v >= rcv
            mv = jnp.where(keep, b1v, rcv)
            mi = jnp.where(keep, b1i, rci)
            hi_m = b0v >= mv
            hv = jnp.where(hi_m, b0v, mv)
            hx = jnp.where(hi_m, b0i, mi)
            lv = jnp.where(hi_m, mv, b0v)
            lx = jnp.where(hi_m, mi, b0i)
            b0v, b0i = plsc.sort_key_val(hv, hx, descending=True)
            b1v, b1i = plsc.sort_key_val(lv, lx, descending=True)
            return b0v, b0i, b1v, b1i

        nch = (cnt + 15) // 16
        b0v, b0i, b1v, b1i = lax.fori_loop(
            0, nch, merge, (ninf16, zero16, ninf16, zero16))

        obase = slot * K_TOT
        outv[pl.ds(obase, 16)] = b0v
        outv[pl.ds(obase + 16, 16)] = b1v
        outi[pl.ds(obase, 16)] = b0i
        outi[pl.ds(obase + 16, 16)] = b1i

    def group(g, _):
        pltpu.sync_copy(neg_hbm.at[pl.ds((row0 + g * G) * N, G * N)], inbuf)

        def row_body(rr, __):
            do_row(rr, g * G + rr)
            return __

        lax.fori_loop(0, G, row_body, 0)
        return _

    lax.fori_loop(0, RPW // G, group, 0)
    pltpu.sync_copy(outv, val_out.at[pl.ds(row0 * K_TOT, RPW * K_TOT)])
    pltpu.sync_copy(outi, idx_out.at[pl.ds(row0 * K_TOT, RPW * K_TOT)])


def kernel(x, batch):
    del batch
    xb = x.reshape(B, N, D)
    neg_adj = _neg_adj(xb).reshape(B * N * N)
    val, nn_idx = _topk_sc(neg_adj)
    val = val.reshape(1, -1)
    start = (jnp.arange(B, dtype=jnp.int32) * N).reshape(B, 1, 1)
    nn_idx = (nn_idx.reshape(B, N, K_TOT) + start).reshape(1, -1)
    center = jnp.repeat(jnp.arange(B * N, dtype=jnp.int32), K_TOT).reshape(1, -1)
    edge_index = jnp.concatenate([nn_idx, center], axis=0)[:, ::DIL]
    return edge_index, val


# speedup vs baseline: 8.3008x; 1.2588x over previous
import functools

import jax
import jax.numpy as jnp
from jax import lax
from jax.experimental import pallas as pl
from jax.experimental.pallas import tpu as pltpu
from jax.experimental.pallas import tpu_sc as plsc

K_TOT = 32
DIL = 2
B = 4
N = 4096
D = 64
QB = 512

NWORK = 32
RPW = (B * N) // NWORK
G = 8
NEG_INF = float("-inf")


def _dist_body(xq_ref, xk_ref, out_ref):
    q = xq_ref[0]
    k = xk_ref[0]
    inner = jax.lax.dot_general(
        q, k, (((1,), (1,)), ((), ())),
        preferred_element_type=jnp.float32,
    )
    x_inner = -2.0 * inner
    qsq = jnp.sum(q * q, axis=-1, keepdims=True)
    ksq = jnp.sum(k * k, axis=-1, keepdims=True)
    adj = qsq + x_inner + ksq.T
    out_ref[...] = jnp.reshape(-adj, (QB * (N // 128), 128))


def _neg_adj(xb):
    return pl.pallas_call(
        _dist_body,
        grid=(B, N // QB),
        in_specs=[
            pl.BlockSpec((1, QB, D), lambda b, i: (b, i, 0)),
            pl.BlockSpec((1, N, D), lambda b, i: (b, 0, 0)),
        ],
        out_specs=pl.BlockSpec(
            (QB * (N // 128), 128), lambda b, i: (b * (N // QB) + i, 0)),
        out_shape=jax.ShapeDtypeStruct((B * N * (N // 128), 128), jnp.float32),
    )(xb, xb)


_MESH = plsc.VectorSubcoreMesh(core_axis_name="c", subcore_axis_name="s")
_VPR = N // 16


@functools.partial(
    pl.kernel,
    out_type=(
        jax.ShapeDtypeStruct((B * N * K_TOT,), jnp.float32),
        jax.ShapeDtypeStruct((B * N * K_TOT,), jnp.int32),
    ),
    mesh=_MESH,
    compiler_params=pltpu.CompilerParams(needs_layout_passes=False),
    scratch_types=[
        pltpu.VMEM((G * N,), jnp.float32),
        pltpu.VMEM((N + 16,), jnp.float32),
        pltpu.VMEM((N + 16,), jnp.int32),
        pltpu.VMEM((RPW * K_TOT,), jnp.float32),
        pltpu.VMEM((RPW * K_TOT,), jnp.int32),
    ],
)
def _topk_sc(neg_hbm, val_out, idx_out, inbuf, cand_v, cand_i, outv, outi):
    cid = lax.axis_index("c")
    sid = lax.axis_index("s")
    wid = sid * 2 + cid
    row0 = wid * RPW
    lane = lax.iota(jnp.int32, 16)
    ninf16 = jnp.full((16,), NEG_INF, jnp.float32)
    zero16 = jnp.zeros((16,), jnp.int32)

    def do_row(rr, slot):
        rbase = rr * N

        UA = 8

        def scan_a(i, carry):
            new = []
            for u in range(UA):
                m1, m2 = carry[2 * u], carry[2 * u + 1]
                v = inbuf[pl.ds(pl.multiple_of(rbase + (i * UA + u) * 16, 16), 16)]
                m2 = jnp.maximum(m2, jnp.minimum(m1, v))
                m1 = jnp.maximum(m1, v)
                new += [m1, m2]
            return tuple(new)

        chains = lax.fori_loop(0, _VPR // UA, scan_a, (ninf16,) * (2 * UA))
        m1, m2 = chains[0], chains[1]
        for u in range(1, UA):
            a1, a2 = chains[2 * u], chains[2 * u + 1]
            m2 = jnp.maximum(jnp.maximum(m2, a2), jnp.minimum(m1, a1))
            m1 = jnp.maximum(m1, a1)
        t = jnp.min(m2)

        UB = 4

        def scan_b(i, off):
            for u in range(UB):
                j = i * UB + u
                v = inbuf[pl.ds(pl.multiple_of(rbase + j * 16, 16), 16)]
                mask = v >= t
                idxv = lane + j * 16
                plsc.store_compressed(cand_v.at[pl.ds(off, 16)], v, mask=mask)
                plsc.store_compressed(cand_i.at[pl.ds(off, 16)], idxv, mask=mask)
                off = off + plsc.all_reduce_population_count(mask)[0]
            return off

        cnt = lax.fori_loop(0, _VPR // UB, scan_b, jnp.int32(0))
        cand_v[pl.ds(cnt, 16)] = ninf16

        def merge(j, carry):
            b0v, b0i, b1v, b1i = carry
            cv = cand_v[pl.ds(pl.multiple_of(j * 16, 16), 16)]
            ci = cand_i[pl.ds(pl.multiple_of(j * 16, 16), 16)]
            cv, ci = plsc.sort_key_val(cv, ci, descending=True)
            rcv = lax.rev(cv, (0,))
            rci = lax.rev(ci, (0,))
            keep = b1v >= rcv
            mv = jnp.where(keep, b1v, rcv)
            mi = jnp.where(keep, b1i, rci)
            hi_m = b0v >= mv
            hv = jnp.where(hi_m, b0v, mv)
            hx = jnp.where(hi_m, b0i, mi)
            lv = jnp.where(hi_m, mv, b0v)
            lx = jnp.where(hi_m, mi, b0i)
            b0v, b0i = plsc.sort_key_val(hv, hx, descending=True)
            b1v, b1i = plsc.sort_key_val(lv, lx, descending=True)
            return b0v, b0i, b1v, b1i

        nch = (cnt + 15) // 16
        b0v, b0i, b1v, b1i = lax.fori_loop(
            0, nch, merge, (ninf16, zero16, ninf16, zero16))

        obase = slot * K_TOT
        outv[pl.ds(obase, 16)] = b0v
        outv[pl.ds(obase + 16, 16)] = b1v
        outi[pl.ds(obase, 16)] = b0i
        outi[pl.ds(obase + 16, 16)] = b1i

    def group(g, _):
        pltpu.sync_copy(neg_hbm.at[pl.ds((row0 + g * G) * N, G * N)], inbuf)

        def row_body(rr, __):
            do_row(rr, g * G + rr)
            return __

        lax.fori_loop(0, G, row_body, 0)
        return _

    lax.fori_loop(0, RPW // G, group, 0)
    pltpu.sync_copy(outv, val_out.at[pl.ds(row0 * K_TOT, RPW * K_TOT)])
    pltpu.sync_copy(outi, idx_out.at[pl.ds(row0 * K_TOT, RPW * K_TOT)])


def kernel(x, batch):
    del batch
    xb = x.reshape(B, N, D)
    neg_adj = _neg_adj(xb).reshape(B * N * N)
    val, nn_idx = _topk_sc(neg_adj)
    val = val.reshape(1, -1)
    start = (jnp.arange(B, dtype=jnp.int32) * N).reshape(B, 1, 1)
    nn_idx = (nn_idx.reshape(B, N, K_TOT) + start).reshape(1, -1)
    center = jnp.repeat(jnp.arange(B * N, dtype=jnp.int32), K_TOT).reshape(1, -1)
    edge_index = jnp.concatenate([nn_idx, center], axis=0)[:, ::DIL]
    return edge_index, val
